# Initial kernel scaffold; baseline (speedup 1.0000x reference)
import jax
import jax.numpy as jnp
from jax.experimental import pallas as pl


def kernel(features, scores, phn_ids):
    raise NotImplementedError("write your pallas kernel here")


# speedup vs baseline: 7.7225x; 7.7225x over previous
import functools

import jax
import jax.numpy as jnp
from jax.experimental import pallas as pl

_NUM_PHN = 39
_KP = 128

_mm = functools.partial(
    jax.lax.dot_general,
    dimension_numbers=(((1,), (0,)), ((), ())),
    preferred_element_type=jnp.float32,
)


def _body(f_ref, sc_ref, phc_ref, phr_ref, out_ref):
    F = f_ref[...]
    sc = sc_ref[...]
    phc = phc_ref[...]
    phr = phr_ref[...]
    N, D = F.shape

    phc_s = jnp.minimum(jnp.maximum(phc, 0), _NUM_PHN - 1)
    phr_s = jnp.minimum(jnp.maximum(phr, 0), _NUM_PHN - 1)
    lane = jax.lax.broadcasted_iota(jnp.int32, (N, _KP), 1)
    sub = jax.lax.broadcasted_iota(jnp.int32, (_KP, N), 0)
    E = (lane == phc_s).astype(jnp.float32)
    ET = (sub == phr_s).astype(jnp.float32)

    valid = sc >= 0.0
    m_high = jnp.where(valid & (sc == 2.0), 1.0, 0.0)
    high_hits = _mm(ET, m_high)
    hpf = jnp.where(high_hits > 0.0, 1.0, 0.0)
    keepf = jnp.where(valid, 1.0, 0.0) * _mm(E, hpf)
    counts = _mm(ET, keepf)
    presentf = jnp.where(counts > 0.0, 1.0, 0.0)
    n_u = jnp.sum(presentf)

    center = _mm(ET, F * keepf) / jnp.maximum(counts, 1.0)
    cn = jnp.sqrt(jnp.sum(center * center, axis=1, keepdims=True))
    center = center / jnp.maximum(cn, 1e-12)
    pn = jnp.sqrt(jnp.sum(center * center, axis=1, keepdims=True))
    p = center / jnp.maximum(pn, 1e-12)

    pn2 = jnp.sum(p * p, axis=1, keepdims=True)
    G = jax.lax.dot_general(
        p, p, (((1,), (1,)), ((), ())), preferred_element_type=jnp.float32
    )
    ii = jax.lax.broadcasted_iota(jnp.int32, (_KP, _KP), 0)
    jj = jax.lax.broadcasted_iota(jnp.int32, (_KP, _KP), 1)
    eye = jnp.where(ii == jj, 1.0, 0.0)
    pn2_row = jnp.sum(G * eye, axis=0, keepdims=True)
    d2 = pn2 + pn2_row - 2.0 * G
    dist = jnp.sqrt(jnp.maximum(d2, 1e-12))
    pair_present = jax.lax.dot_general(
        presentf, presentf, (((1,), (1,)), ((), ())),
        preferred_element_type=jnp.float32,
    )
    pair_mask = (pair_present > 0.5) & (ii < jj)
    denom = jnp.maximum(n_u * (n_u - 1.0) * 0.5, 1.0)
    diversity = jnp.sum(jnp.where(pair_mask, dist, 0.0)) / denom

    fn2 = jnp.sum(F * F, axis=1, keepdims=True)
    fhat = F / jnp.maximum(jnp.sqrt(fn2), 1e-12)
    Cg = _mm(E, p)
    diff = fhat - Cg
    dsq = jnp.sum(diff * diff, axis=1, keepdims=True)
    nzf = keepf * jnp.where(dsq > 0.0, 1.0, 0.0)
    cnt = jnp.sum(nzf)
    w = 3.0 - sc
    tsum = jnp.sum(nzf * jnp.sqrt(dsq) * w)
    tightness = tsum / jnp.maximum(cnt, 1.0)

    loss = 0.1 * tightness - 0.5 * diversity
    out_ref[...] = jnp.broadcast_to(jnp.where(n_u >= 2.0, loss, 0.0), (1, 1))


def kernel(features, scores, phn_ids):
    B, T, D = features.shape
    N = B * T
    F = features.reshape(N, D)
    sc = scores.reshape(N, 1)
    phc = phn_ids.reshape(N, 1).astype(jnp.int32)
    phr = phn_ids.reshape(1, N).astype(jnp.int32)
    out = pl.pallas_call(
        _body,
        out_shape=jax.ShapeDtypeStruct((1, 1), jnp.float32),
    )(F, sc, phc, phr)
    return out[0, 0]
